# Initial kernel scaffold; baseline (speedup 1.0000x reference)
import jax
import jax.numpy as jnp
from jax.experimental import pallas as pl


def kernel(duration, returns, direction, start_at, stop_at, batch_idx, market):
    raise NotImplementedError("write your pallas kernel here")


# speedup vs baseline: 32.7654x; 32.7654x over previous
import functools

import jax
import jax.numpy as jnp
from jax import lax
from jax.experimental import pallas as pl
from jax.experimental.pallas import tpu as pltpu
from jax.experimental.pallas import tpu_sc as plsc

_LOW_D = 10.0
_HIGH_D = 30.0
_HIGH_R = 0.01
_LOW_R = 0.005


def _precompute_body(dur, ret, dirn, start, stop, bat, mar, fs_o, fp_o, cat_o,
                     w_o, *, M, BM):
    d = dur[...]
    r = ret[...]
    di = dirn[...]
    dscore = jnp.where(d > _HIGH_D, 1.0, jnp.where(d < _LOW_D, 0.5, 0.75))
    rscore = jnp.where(r > _HIGH_R, 1.0, jnp.where(r < _LOW_R, 0.0, 0.75))
    score = (dscore * rscore).astype(jnp.float32)
    side = (r < _LOW_R) | ((d > _HIGH_D) & (r < _HIGH_R))
    not_side = jnp.logical_not(side)
    up = not_side & (di > 0)
    down = not_side & (di < 0)
    cat = jnp.where(up, 0, jnp.where(side, 1, jnp.where(down, 2, 3)))
    cat_o[...] = cat.astype(jnp.int32)
    w_o[...] = jnp.where(cat != 3, score, 0.0)
    col = bat[...] * M + mar[...]
    fs_o[...] = start[...] * BM + col
    fp_o[...] = stop[...] * BM + col


def _precompute(duration, returns, direction, start_at, stop_at, batch_idx,
                market, M, BM):
    n = duration.shape[0]
    assert n % 128 == 0
    rows = n // 128
    br = rows
    for cand in (512, 256, 128, 64, 8):
        if rows % cand == 0:
            br = cand
            break
    g = rows // br
    shp = (rows, 128)
    ins = [x.reshape(shp) for x in (duration, returns, direction, start_at,
                                    stop_at, batch_idx, market)]
    spec = pl.BlockSpec((br, 128), lambda i: (i, 0))
    fs, fp, cat, w = pl.pallas_call(
        functools.partial(_precompute_body, M=M, BM=BM),
        grid=(g,),
        in_specs=[spec] * 7,
        out_specs=[spec] * 4,
        out_shape=[
            jax.ShapeDtypeStruct(shp, jnp.int32),
            jax.ShapeDtypeStruct(shp, jnp.int32),
            jax.ShapeDtypeStruct(shp, jnp.int32),
            jax.ShapeDtypeStruct(shp, jnp.float32),
        ],
    )(*ins)
    return fs.reshape(n), fp.reshape(n), cat.reshape(n), w.reshape(n)


def _sc_scatter(fs, fp, cat, w, T, BM, n_e=2048, trash=16384):
    npts = T * BM
    half = npts // 2
    npad = fs.shape[0]
    ev_per_tile = npad // 16
    nblk = ev_per_tile // n_e
    assert ev_per_tile % n_e == 0
    zw = half // 16
    zchunk = 8192
    assert zw % zchunk == 0
    acc_words = half + trash
    mesh = plsc.VectorSubcoreMesh(core_axis_name="c", subcore_axis_name="s")

    def body(fs_hbm, fp_hbm, cat_hbm, w_hbm, acc_hbm, fs_v, fp_v, cat_v, w_v,
             sidx_v, sval_v, zero_v, acc_sh):
        core = lax.axis_index("c")
        tid = lax.axis_index("s")
        half_off = core * half
        lanes = lax.iota(jnp.int32, 16)

        def zinit(i, _):
            zero_v[pl.ds(i * 16, 16)] = jnp.zeros((16,), jnp.float32)
            return _

        lax.fori_loop(0, zchunk // 16, zinit, None)

        for c_ch in range(4):
            def zblk(j, _):
                pltpu.sync_copy(zero_v,
                                acc_sh.at[pl.ds(tid * zw + j * zchunk, zchunk)])
                return _

            lax.fori_loop(0, zw // zchunk, zblk, None)
            plsc.subcore_barrier()

            def blk(bi, _, c_ch=c_ch):
                base = tid * ev_per_tile + bi * n_e
                pltpu.sync_copy(fs_hbm.at[pl.ds(base, n_e)], fs_v)
                pltpu.sync_copy(fp_hbm.at[pl.ds(base, n_e)], fp_v)
                if c_ch < 3:
                    pltpu.sync_copy(cat_hbm.at[pl.ds(base, n_e)], cat_v)
                else:
                    pltpu.sync_copy(w_hbm.at[pl.ds(base, n_e)], w_v)

                def lane(i, _, c_ch=c_ch):
                    s = fs_v[pl.ds(i * 16, 16)] - half_off
                    p = fp_v[pl.ds(i * 16, 16)] - half_off
                    tr = half + ((lanes + i * 16) & (trash - 1))
                    si = jnp.where((s >= 0) & (s < half), s, tr)
                    pi = jnp.where((p >= 0) & (p < half), p, tr)
                    if c_ch < 3:
                        v = jnp.where(cat_v[pl.ds(i * 16, 16)] == c_ch,
                                      jnp.float32(1.0), jnp.float32(0.0))
                    else:
                        v = w_v[pl.ds(i * 16, 16)]
                    sidx_v[pl.ds(i * 16, 16)] = si
                    sidx_v[pl.ds(n_e + i * 16, 16)] = pi
                    sval_v[pl.ds(i * 16, 16)] = v
                    sval_v[pl.ds(n_e + i * 16, 16)] = -v
                    return _

                lax.fori_loop(0, n_e // 16, lane, None)
                pltpu.sync_copy(sval_v, acc_sh.at[sidx_v], add=True)
                return _

            lax.fori_loop(0, nblk, blk, None)
            plsc.subcore_barrier()
            pltpu.sync_copy(
                acc_sh.at[pl.ds(tid * zw, zw)],
                acc_hbm.at[c_ch, pl.ds(half_off + tid * zw, zw)])
            plsc.subcore_barrier()

    run = pl.kernel(
        body,
        out_type=jax.ShapeDtypeStruct((4, npts), jnp.float32),
        mesh=mesh,
        scratch_types=[
            pltpu.VMEM((n_e,), jnp.int32),
            pltpu.VMEM((n_e,), jnp.int32),
            pltpu.VMEM((n_e,), jnp.int32),
            pltpu.VMEM((n_e,), jnp.float32),
            pltpu.VMEM((2 * n_e,), jnp.int32),
            pltpu.VMEM((2 * n_e,), jnp.float32),
            pltpu.VMEM((zchunk,), jnp.float32),
            pltpu.VMEM_SHARED((acc_words,), jnp.float32),
        ],
    )
    return run(fs, fp, cat, w)


def _cumsum_body(acc_ref, out_ref, carry_ref, *, BT):
    @pl.when(pl.program_id(1) == 0)
    def _():
        carry_ref[...] = jnp.zeros_like(carry_ref)

    blk = acc_ref[0]
    row = lax.broadcasted_iota(jnp.int32, (BT, BT), 0)
    col = lax.broadcasted_iota(jnp.int32, (BT, BT), 1)
    tri = (row >= col).astype(jnp.float32)
    cum = jax.lax.dot(tri, blk, preferred_element_type=jnp.float32)
    out_ref[0] = cum + carry_ref[...]
    carry_ref[...] = carry_ref[...] + cum[BT - 1:BT, :]


def _cumsum(acc, T, BM, BT=512):
    acc3 = acc.reshape(4, T, BM)
    spec = pl.BlockSpec((1, BT, BM), lambda c, t: (c, t, 0))
    return pl.pallas_call(
        functools.partial(_cumsum_body, BT=BT),
        grid=(4, T // BT),
        in_specs=[spec],
        out_specs=spec,
        out_shape=jax.ShapeDtypeStruct((4, T, BM), jnp.float32),
        scratch_shapes=[pltpu.VMEM((1, BM), jnp.float32)],
    )(acc3)


def kernel(duration, returns, direction, start_at, stop_at, batch_idx, market):
    n = duration.shape[0]
    T = 4096
    B = 8
    M = 64
    BM = B * M
    n_e = 2048
    grp = 16 * n_e
    npad = ((n + grp - 1) // grp) * grp
    pad = npad - n
    if pad:
        spread = (jnp.arange(pad, dtype=jnp.int32) * 7) % T
        duration = jnp.concatenate([duration, jnp.zeros((pad,), jnp.float32)])
        returns = jnp.concatenate([returns, jnp.ones((pad,), jnp.float32)])
        direction = jnp.concatenate([direction, jnp.zeros((pad,), jnp.float32)])
        start_at = jnp.concatenate([start_at, spread])
        stop_at = jnp.concatenate([stop_at, spread])
        batch_idx = jnp.concatenate([batch_idx, jnp.zeros((pad,), jnp.int32)])
        market = jnp.concatenate([market, jnp.zeros((pad,), jnp.int32)])
    fs, fp, cat, w = _precompute(duration, returns, direction, start_at,
                                 stop_at, batch_idx, market, M, BM)
    acc = _sc_scatter(fs, fp, cat, w, T, BM, n_e=n_e)
    cum = _cumsum(acc, T, BM)
    cats = jnp.transpose(cum[:3], (1, 2, 0)).reshape(T, B, M, 3)
    mask = cum[3].reshape(T, B, M)
    return cats, mask


# speedup vs baseline: 76.3384x; 2.3298x over previous
import functools

import jax
import jax.numpy as jnp
from jax import lax
from jax.experimental import pallas as pl
from jax.experimental.pallas import tpu as pltpu
from jax.experimental.pallas import tpu_sc as plsc

_LOW_D = 10.0
_HIGH_D = 30.0
_HIGH_R = 0.01
_LOW_R = 0.005


def _precompute_body(dur, ret, dirn, start, stop, bat, mar, fs_o, fp_o, cat_o,
                     w_o, *, M, BM):
    d = dur[...]
    r = ret[...]
    di = dirn[...]
    dscore = jnp.where(d > _HIGH_D, 1.0, jnp.where(d < _LOW_D, 0.5, 0.75))
    rscore = jnp.where(r > _HIGH_R, 1.0, jnp.where(r < _LOW_R, 0.0, 0.75))
    score = (dscore * rscore).astype(jnp.float32)
    side = (r < _LOW_R) | ((d > _HIGH_D) & (r < _HIGH_R))
    not_side = jnp.logical_not(side)
    up = not_side & (di > 0)
    down = not_side & (di < 0)
    cat = jnp.where(up, 0, jnp.where(side, 1, jnp.where(down, 2, 3)))
    cat_o[...] = cat.astype(jnp.int32)
    w_o[...] = jnp.where(cat != 3, score, 0.0)
    col = bat[...] * M + mar[...]
    fs_o[...] = start[...] * BM + col
    fp_o[...] = stop[...] * BM + col


def _precompute(duration, returns, direction, start_at, stop_at, batch_idx,
                market, M, BM):
    n = duration.shape[0]
    assert n % 128 == 0
    rows = n // 128
    br = rows
    for cand in (512, 256, 128, 64, 8):
        if rows % cand == 0:
            br = cand
            break
    g = rows // br
    shp = (rows, 128)
    ins = [x.reshape(shp) for x in (duration, returns, direction, start_at,
                                    stop_at, batch_idx, market)]
    spec = pl.BlockSpec((br, 128), lambda i: (i, 0))
    fs, fp, cat, w = pl.pallas_call(
        functools.partial(_precompute_body, M=M, BM=BM),
        grid=(g,),
        in_specs=[spec] * 7,
        out_specs=[spec] * 4,
        out_shape=[
            jax.ShapeDtypeStruct(shp, jnp.int32),
            jax.ShapeDtypeStruct(shp, jnp.int32),
            jax.ShapeDtypeStruct(shp, jnp.int32),
            jax.ShapeDtypeStruct(shp, jnp.float32),
        ],
    )(*ins)
    return fs.reshape(n), fp.reshape(n), cat.reshape(n), w.reshape(n)


def _sc_scatter(fs, fp, cat, w, T, BM, n_e=1024, trash=8192):
    npts = T * BM
    quar = npts // 4
    half = npts // 2
    cat_words = quar * 3
    acc_words = cat_words + trash
    npad = fs.shape[0]
    ev_per_tile = npad // 16
    nblk = ev_per_tile // n_e
    assert ev_per_tile % n_e == 0 and nblk % 2 == 0 and n_e % 16 == 0
    zchunk = 4096
    zw_cat = cat_words // 16
    zw_w = half // 16
    assert zw_cat % zchunk == 0 and zw_w % zchunk == 0
    mesh = plsc.VectorSubcoreMesh(core_axis_name="c", subcore_axis_name="s")

    def body(fs_hbm, fp_hbm, cat_hbm, w_hbm, cats_hbm, w_out_hbm,
             fs_v0, fs_v1, fp_v0, fp_v1, cat_v0, cat_v1, w_v0, w_v1,
             sidx0, sidx1, sval0, sval1, cval, zero_v,
             sem_st0, sem_st1, sem_sc0, sem_sc1, acc_sh):
        core = lax.axis_index("c")
        tid = lax.axis_index("s")
        lanes = lax.iota(jnp.int32, 16)
        stage = ((fs_v0, fp_v0, cat_v0, w_v0, sidx0, sval0, sem_st0, sem_sc0),
                 (fs_v1, fp_v1, cat_v1, w_v1, sidx1, sval1, sem_st1, sem_sc1))

        def zinit(i, _):
            zero_v[pl.ds(i * 16, 16)] = jnp.zeros((16,), jnp.float32)
            return _

        lax.fori_loop(0, zchunk // 16, zinit, None)

        def cinit(i, _):
            cval[pl.ds(i * 16, 16)] = jnp.full((16,), 1.0, jnp.float32)
            cval[pl.ds(n_e + i * 16, 16)] = jnp.full((16,), -1.0, jnp.float32)
            return _

        lax.fori_loop(0, n_e // 16, cinit, None)

        def run_scan(is_cat, piece, out_hbm, out_off, zw):
            def zblk(j, _):
                pltpu.sync_copy(zero_v,
                                acc_sh.at[pl.ds(tid * zw + j * zchunk, zchunk)])
                return _

            lax.fori_loop(0, zw // zchunk, zblk, None)
            plsc.subcore_barrier()

            def st_descs(bi, p):
                base = tid * ev_per_tile + bi * n_e
                fs_v, fp_v, cat_v, w_v, _, _, sem_st, _ = stage[p]
                d = [pltpu.make_async_copy(fs_hbm.at[pl.ds(base, n_e)], fs_v,
                                           sem_st),
                     pltpu.make_async_copy(fp_hbm.at[pl.ds(base, n_e)], fp_v,
                                           sem_st)]
                if is_cat:
                    d.append(pltpu.make_async_copy(
                        cat_hbm.at[pl.ds(base, n_e)], cat_v, sem_st))
                else:
                    d.append(pltpu.make_async_copy(
                        w_hbm.at[pl.ds(base, n_e)], w_v, sem_st))
                return d

            def sc_desc(p):
                _, _, _, _, sidx, sval, _, sem_sc = stage[p]
                src = cval if is_cat else sval
                return pltpu.make_async_copy(src, acc_sh.at[sidx], sem_sc)

            def fire_scat(p):
                _, _, _, _, sidx, sval, _, sem_sc = stage[p]
                src = cval if is_cat else sval
                pltpu.async_copy(src, acc_sh.at[sidx], sem_sc, add=True)

            def compute(p):
                fs_v, fp_v, cat_v, w_v, sidx, sval, _, _ = stage[p]

                def lane(i, _):
                    s = fs_v[pl.ds(i * 16, 16)] - piece * (quar if is_cat
                                                           else half)
                    q = fp_v[pl.ds(i * 16, 16)] - piece * (quar if is_cat
                                                           else half)
                    tr = cat_words + ((lanes + i * 16) & (trash - 1))
                    if is_cat:
                        c = cat_v[pl.ds(i * 16, 16)]
                        oks = (s >= 0) & (s < quar) & (c < 3)
                        okq = (q >= 0) & (q < quar) & (c < 3)
                        si = jnp.where(oks, s * 3 + c, tr)
                        qi = jnp.where(okq, q * 3 + c, tr)
                    else:
                        si = jnp.where((s >= 0) & (s < half), s, tr)
                        qi = jnp.where((q >= 0) & (q < half), q, tr)
                        v = w_v[pl.ds(i * 16, 16)]
                        sval[pl.ds(i * 16, 16)] = v
                        sval[pl.ds(n_e + i * 16, 16)] = -v
                    sidx[pl.ds(i * 16, 16)] = si
                    sidx[pl.ds(n_e + i * 16, 16)] = qi
                    return _

                lax.fori_loop(0, n_e // 16, lane, None)

            for dd in st_descs(0, 0):
                dd.start()

            def body2(j, _):
                b0 = 2 * j
                for dd in st_descs(b0, 0):
                    dd.wait()

                for dd in st_descs(b0 + 1, 1):
                    dd.start()

                @pl.when(j > 0)
                def _():
                    sc_desc(0).wait()

                compute(0)
                fire_scat(0)
                for dd in st_descs(b0 + 1, 1):
                    dd.wait()

                @pl.when(j + 1 < nblk // 2)
                def _():
                    for dd in st_descs(b0 + 2, 0):
                        dd.start()

                @pl.when(j > 0)
                def _():
                    sc_desc(1).wait()

                compute(1)
                fire_scat(1)
                return _

            lax.fori_loop(0, nblk // 2, body2, None)
            sc_desc(0).wait()
            sc_desc(1).wait()
            plsc.subcore_barrier()
            pltpu.sync_copy(acc_sh.at[pl.ds(tid * zw, zw)],
                            out_hbm.at[pl.ds(out_off + tid * zw, zw)])
            plsc.subcore_barrier()

        for jq in range(2):
            qq = core * 2 + jq
            run_scan(True, qq, cats_hbm, qq * cat_words, zw_cat)
        run_scan(False, core, w_out_hbm, core * half, zw_w)

    run = pl.kernel(
        body,
        out_type=(jax.ShapeDtypeStruct((npts * 3,), jnp.float32),
                  jax.ShapeDtypeStruct((npts,), jnp.float32)),
        mesh=mesh,
        scratch_types=[
            pltpu.VMEM((n_e,), jnp.int32),
            pltpu.VMEM((n_e,), jnp.int32),
            pltpu.VMEM((n_e,), jnp.int32),
            pltpu.VMEM((n_e,), jnp.int32),
            pltpu.VMEM((n_e,), jnp.int32),
            pltpu.VMEM((n_e,), jnp.int32),
            pltpu.VMEM((n_e,), jnp.float32),
            pltpu.VMEM((n_e,), jnp.float32),
            pltpu.VMEM((2 * n_e,), jnp.int32),
            pltpu.VMEM((2 * n_e,), jnp.int32),
            pltpu.VMEM((2 * n_e,), jnp.float32),
            pltpu.VMEM((2 * n_e,), jnp.float32),
            pltpu.VMEM((2 * n_e,), jnp.float32),
            pltpu.VMEM((zchunk,), jnp.float32),
            pltpu.SemaphoreType.DMA,
            pltpu.SemaphoreType.DMA,
            pltpu.SemaphoreType.DMA,
            pltpu.SemaphoreType.DMA,
            pltpu.VMEM_SHARED((acc_words,), jnp.float32),
        ],
    )
    return run(fs, fp, cat, w)


def _cumsum_body(acc_ref, out_ref, carry_ref, *, BT):
    @pl.when(pl.program_id(1) == 0)
    def _():
        carry_ref[...] = jnp.zeros_like(carry_ref)

    blk = acc_ref[0]
    row = lax.broadcasted_iota(jnp.int32, (BT, BT), 0)
    col = lax.broadcasted_iota(jnp.int32, (BT, BT), 1)
    tri = (row >= col).astype(jnp.float32)
    cum = jax.lax.dot(tri, blk, preferred_element_type=jnp.float32)
    out_ref[0] = cum + carry_ref[...]
    carry_ref[...] = carry_ref[...] + cum[BT - 1:BT, :]


def _cumsum(acc, T, C, BT=512):
    acc2 = acc.reshape(1, T, C)
    spec = pl.BlockSpec((1, BT, C), lambda c, t: (c, t, 0))
    out = pl.pallas_call(
        functools.partial(_cumsum_body, BT=BT),
        grid=(1, T // BT),
        in_specs=[spec],
        out_specs=spec,
        out_shape=jax.ShapeDtypeStruct((1, T, C), jnp.float32),
        scratch_shapes=[pltpu.VMEM((1, C), jnp.float32)],
    )(acc2)
    return out.reshape(T, C)


def kernel(duration, returns, direction, start_at, stop_at, batch_idx, market):
    n = duration.shape[0]
    T = 4096
    B = 8
    M = 64
    BM = B * M
    n_e = 1024
    grp = 16 * 2 * n_e
    npad = ((n + grp - 1) // grp) * grp
    pad = npad - n
    if pad:
        spread = (jnp.arange(pad, dtype=jnp.int32) * 7) % T
        duration = jnp.concatenate([duration, jnp.zeros((pad,), jnp.float32)])
        returns = jnp.concatenate([returns, jnp.ones((pad,), jnp.float32)])
        direction = jnp.concatenate([direction, jnp.zeros((pad,), jnp.float32)])
        start_at = jnp.concatenate([start_at, spread])
        stop_at = jnp.concatenate([stop_at, spread])
        batch_idx = jnp.concatenate([batch_idx, jnp.zeros((pad,), jnp.int32)])
        market = jnp.concatenate([market, jnp.zeros((pad,), jnp.int32)])
    fs, fp, cat, w = _precompute(duration, returns, direction, start_at,
                                 stop_at, batch_idx, market, M, BM)
    acc_cats, acc_w = _sc_scatter(fs, fp, cat, w, T, BM, n_e=n_e)
    cats = _cumsum(acc_cats, T, BM * 3).reshape(T, B, M, 3)
    mask = _cumsum(acc_w, T, BM).reshape(T, B, M)
    return cats, mask
